# Initial kernel scaffold; baseline (speedup 1.0000x reference)
import jax
import jax.numpy as jnp
from jax.experimental import pallas as pl


def kernel(feat, edge_index, W1, b1, W2, b2):
    raise NotImplementedError("write your pallas kernel here")


# speedup vs baseline: 8.7724x; 8.7724x over previous
import functools

import jax
import jax.numpy as jnp
from jax import lax
from jax.experimental import pallas as pl
from jax.experimental.pallas import tpu as pltpu
from jax.experimental.pallas import tpu_sc as plsc

N = 10000
E = 320000
D_IN = 128
D_H = 128
D_OUT = 64

NC = 2
NS = 16
B = 128
N_PAD = 10240
ROWS_PER_TILE = N_PAD // NS
E_PAD = 327680
NCHUNK = E_PAD // B
RB = 1024

_mesh = plsc.VectorSubcoreMesh(
    core_axis_name="c", subcore_axis_name="s", num_cores=NC, num_subcores=NS
)


def _deg_body(ei_hbm, zcol_hbm, deg_hbm, idx_all, ones_v, hist_sh):
    c = lax.axis_index("c")
    s = lax.axis_index("s")
    pltpu.sync_copy(
        zcol_hbm.at[pl.ds(s * ROWS_PER_TILE, ROWS_PER_TILE)],
        hist_sh.at[pl.ds(s * ROWS_PER_TILE, ROWS_PER_TILE)],
    )
    for k in range(B // 16):
        ones_v[pl.ds(k * 16, 16)] = jnp.full((16,), 1.0, jnp.float32)
    per_tile = NCHUNK // NS
    pltpu.sync_copy(ei_hbm.at[c, pl.ds(s * per_tile, per_tile)], idx_all)
    plsc.subcore_barrier()

    @pl.loop(0, per_tile)
    def _chunk(t):
        pltpu.sync_copy(ones_v, hist_sh.at[idx_all.at[t]], add=True)

    plsc.subcore_barrier()
    pltpu.sync_copy(
        hist_sh.at[pl.ds(s * ROWS_PER_TILE, ROWS_PER_TILE)],
        deg_hbm.at[c, pl.ds(s * ROWS_PER_TILE, ROWS_PER_TILE)],
    )


_deg_kernel = pl.kernel(
    _deg_body,
    out_type=jax.ShapeDtypeStruct((NC, N_PAD), jnp.float32),
    mesh=_mesh,
    scratch_types=[
        pltpu.VMEM((NCHUNK // NS, B), jnp.int32),
        pltpu.VMEM((B,), jnp.float32),
        pltpu.VMEM_SHARED((N_PAD,), jnp.float32),
    ],
)


def _prop_body(width, tab_hbm, ei_hbm, zer_hbm, out_hbm,
               idx_src, idx_dst, rows_v, acc_sh):
    c = lax.axis_index("c")
    s = lax.axis_index("s")
    pltpu.sync_copy(
        zer_hbm.at[pl.ds(s * ROWS_PER_TILE, ROWS_PER_TILE), :],
        acc_sh.at[pl.ds(s * ROWS_PER_TILE, ROWS_PER_TILE), :],
    )
    per_tile = NCHUNK // (NC * NS)
    base = c * (NCHUNK // NC) + s * per_tile
    pltpu.sync_copy(ei_hbm.at[0, pl.ds(base, per_tile)], idx_src)
    pltpu.sync_copy(ei_hbm.at[1, pl.ds(base, per_tile)], idx_dst)
    plsc.subcore_barrier()

    @pl.loop(0, per_tile)
    def _chunk(t):
        pltpu.sync_copy(tab_hbm.at[idx_src.at[t]], rows_v)
        pltpu.sync_copy(rows_v, acc_sh.at[idx_dst.at[t]], add=True)

    plsc.subcore_barrier()
    pltpu.sync_copy(
        acc_sh.at[pl.ds(s * ROWS_PER_TILE, ROWS_PER_TILE), :],
        out_hbm.at[c, pl.ds(s * ROWS_PER_TILE, ROWS_PER_TILE), :],
    )


def _make_prop_kernel(width):
    per_tile = NCHUNK // (NC * NS)
    return pl.kernel(
        functools.partial(_prop_body, width),
        out_type=jax.ShapeDtypeStruct((NC, N_PAD, width), jnp.float32),
        mesh=_mesh,
        scratch_types=[
            pltpu.VMEM((per_tile, B), jnp.int32),
            pltpu.VMEM((per_tile, B), jnp.int32),
            pltpu.VMEM((B, width), jnp.float32),
            pltpu.VMEM_SHARED((N_PAD, width), jnp.float32),
        ],
    )


_prop128 = _make_prop_kernel(D_H)


def _k2_body(feat_ref, w1_ref, deg_ref, y_ref):
    rs_out = lax.rsqrt(jnp.maximum(deg_ref[0], 1.0))
    y = jnp.dot(feat_ref[...], w1_ref[...], preferred_element_type=jnp.float32)
    y_ref[...] = y * rs_out


def _k4_body(p_ref, deg_ref, b1_ref, w2_ref, z_ref):
    p = p_ref[0] + p_ref[1]
    rs_in = lax.rsqrt(jnp.maximum(deg_ref[1], 1.0))
    rs_out = lax.rsqrt(jnp.maximum(deg_ref[0], 1.0))
    h1 = jnp.maximum(p * rs_in + b1_ref[...], 0.0)
    z = jnp.dot(h1 * rs_out, w2_ref[...], preferred_element_type=jnp.float32)
    z_ref[...] = jnp.concatenate([z, jnp.zeros_like(z)], axis=1)


def _k6_body(p_ref, deg_ref, b2_ref, o_ref):
    p = p_ref[0, :, :D_OUT] + p_ref[1, :, :D_OUT]
    rs_in = lax.rsqrt(jnp.maximum(deg_ref[1], 1.0))
    o_ref[...] = p * rs_in + b2_ref[...]


_GRID = N_PAD // RB

_k2 = pl.pallas_call(
    _k2_body,
    grid=(_GRID,),
    in_specs=[
        pl.BlockSpec((RB, D_IN), lambda i: (i, 0)),
        pl.BlockSpec((D_IN, D_H), lambda i: (0, 0)),
        pl.BlockSpec((NC, RB, 1), lambda i: (0, i, 0)),
    ],
    out_specs=pl.BlockSpec((RB, D_H), lambda i: (i, 0)),
    out_shape=jax.ShapeDtypeStruct((N_PAD, D_H), jnp.float32),
)

_k4 = pl.pallas_call(
    _k4_body,
    grid=(_GRID,),
    in_specs=[
        pl.BlockSpec((NC, RB, D_H), lambda i: (0, i, 0)),
        pl.BlockSpec((NC, RB, 1), lambda i: (0, i, 0)),
        pl.BlockSpec((1, D_H), lambda i: (0, 0)),
        pl.BlockSpec((D_H, D_OUT), lambda i: (0, 0)),
    ],
    out_specs=pl.BlockSpec((RB, D_H), lambda i: (i, 0)),
    out_shape=jax.ShapeDtypeStruct((N_PAD, D_H), jnp.float32),
)

_k6 = pl.pallas_call(
    _k6_body,
    grid=(_GRID,),
    in_specs=[
        pl.BlockSpec((NC, RB, D_H), lambda i: (0, i, 0)),
        pl.BlockSpec((NC, RB, 1), lambda i: (0, i, 0)),
        pl.BlockSpec((1, D_OUT), lambda i: (0, 0)),
    ],
    out_specs=pl.BlockSpec((RB, D_OUT), lambda i: (i, 0)),
    out_shape=jax.ShapeDtypeStruct((N_PAD, D_OUT), jnp.float32),
)


@jax.jit
def kernel(feat, edge_index, W1, b1, W2, b2):
    feat_pad = jnp.zeros((N_PAD, D_IN), jnp.float32).at[:N].set(feat)
    npad = E_PAD - E
    pad_idx = (N + (jnp.arange(npad, dtype=jnp.int32) % (N_PAD - N)))
    ei_pad = jnp.concatenate(
        [edge_index, jnp.broadcast_to(pad_idx, (2, npad))], axis=1
    ).reshape(2, NCHUNK, B)

    zcol = jnp.zeros((N_PAD,), jnp.float32)
    zeros128 = jnp.zeros((N_PAD, D_H), jnp.float32)

    deg = _deg_kernel(ei_pad, zcol)
    deg3 = deg.reshape(NC, N_PAD, 1)

    y = _k2(feat_pad, W1, deg3)
    p1 = _prop128(y, ei_pad, zeros128)
    z = _k4(p1, deg3, b1.reshape(1, D_H), W2)
    p2 = _prop128(z, ei_pad, zeros128)
    out = _k6(p2, deg3, b2.reshape(1, D_OUT))
    return out[:N]


# speedup vs baseline: 12.5093x; 1.4260x over previous
import functools

import jax
import jax.numpy as jnp
from jax import lax
from jax.experimental import pallas as pl
from jax.experimental.pallas import tpu as pltpu
from jax.experimental.pallas import tpu_sc as plsc

N = 10000
E = 320000
D_IN = 128
D_H = 128
D_OUT = 64

NC = 2
NS = 16
B = 128
N_PAD = 10240
ROWS_PER_TILE = N_PAD // NS
E_PAD = 327680
NCHUNK = E_PAD // B
RB = 1024

_mesh = plsc.VectorSubcoreMesh(
    core_axis_name="c", subcore_axis_name="s", num_cores=NC, num_subcores=NS
)


def _deg_body(ei_hbm, zcol_hbm, deg_hbm, idx_all, ones_v, hist_sh, sem_s):
    c = lax.axis_index("c")
    s = lax.axis_index("s")
    pltpu.sync_copy(
        zcol_hbm.at[pl.ds(s * ROWS_PER_TILE, ROWS_PER_TILE)],
        hist_sh.at[pl.ds(s * ROWS_PER_TILE, ROWS_PER_TILE)],
    )
    for k in range(B // 16):
        ones_v[pl.ds(k * 16, 16)] = jnp.full((16,), 1.0, jnp.float32)
    per_tile = NCHUNK // NS
    pltpu.sync_copy(ei_hbm.at[c, pl.ds(s * per_tile, per_tile)], idx_all)
    plsc.subcore_barrier()

    GRP = 16

    @pl.loop(0, per_tile // GRP)
    def _grp(g):
        for j in range(GRP):
            pltpu.async_copy(ones_v, hist_sh.at[idx_all.at[g * GRP + j]],
                             sem_s, add=True)
        for j in range(GRP):
            pltpu.make_async_copy(
                ones_v, hist_sh.at[idx_all.at[g * GRP + j]], sem_s).wait()

    plsc.subcore_barrier()
    pltpu.sync_copy(
        hist_sh.at[pl.ds(s * ROWS_PER_TILE, ROWS_PER_TILE)],
        deg_hbm.at[c, pl.ds(s * ROWS_PER_TILE, ROWS_PER_TILE)],
    )


_deg_kernel = pl.kernel(
    _deg_body,
    out_type=jax.ShapeDtypeStruct((NC, N_PAD), jnp.float32),
    mesh=_mesh,
    scratch_types=[
        pltpu.VMEM((NCHUNK // NS, B), jnp.int32),
        pltpu.VMEM((B,), jnp.float32),
        pltpu.VMEM_SHARED((N_PAD,), jnp.float32),
        pltpu.SemaphoreType.DMA,
    ],
)


def _prop_body(width, tab_hbm, ei_hbm, zer_hbm, out_hbm,
               idx_src, idx_dst, rows_v, acc_sh,
               sem_g0, sem_g1, sem_s0, sem_s1):
    c = lax.axis_index("c")
    s = lax.axis_index("s")
    pltpu.sync_copy(
        zer_hbm.at[pl.ds(s * ROWS_PER_TILE, ROWS_PER_TILE), :],
        acc_sh.at[pl.ds(s * ROWS_PER_TILE, ROWS_PER_TILE), :],
    )
    per_tile = NCHUNK // (NC * NS)
    half = per_tile // 2
    base = c * (NCHUNK // NC) + s * per_tile

    def gather(t, b, sem):
        pltpu.async_copy(tab_hbm.at[idx_src.at[t]], rows_v.at[b], sem)

    def gather_wait(t, b, sem):
        pltpu.make_async_copy(tab_hbm.at[idx_src.at[t]], rows_v.at[b],
                              sem).wait()

    def scat(t, b, sem):
        pltpu.async_copy(rows_v.at[b], acc_sh.at[idx_dst.at[t]], sem,
                         add=True)

    def scat_wait(t, b, sem):
        pltpu.make_async_copy(rows_v.at[b], acc_sh.at[idx_dst.at[t]],
                              sem).wait()

    plsc.subcore_barrier()
    npairs = half // 2

    @pl.loop(0, 2)
    def _half(h):
        pltpu.sync_copy(ei_hbm.at[0, pl.ds(base + h * half, half)], idx_src)
        pltpu.sync_copy(ei_hbm.at[1, pl.ds(base + h * half, half)], idx_dst)
        gather(0, 0, sem_g0)
        gather(1, 1, sem_g1)

        @pl.loop(0, npairs)
        def _pair(i):
            t0 = 2 * i
            t1 = t0 + 1
            gather_wait(t0, 0, sem_g0)
            scat(t0, 0, sem_s0)
            gather_wait(t1, 1, sem_g1)
            scat_wait(t0, 0, sem_s0)

            @pl.when(i + 1 < npairs)
            def _():
                gather(t0 + 2, 0, sem_g0)

            scat(t1, 1, sem_s1)
            scat_wait(t1, 1, sem_s1)

            @pl.when(i + 1 < npairs)
            def _():
                gather(t1 + 2, 1, sem_g1)

    plsc.subcore_barrier()
    pltpu.sync_copy(
        acc_sh.at[pl.ds(s * ROWS_PER_TILE, ROWS_PER_TILE), :],
        out_hbm.at[c, pl.ds(s * ROWS_PER_TILE, ROWS_PER_TILE), :],
    )


def _make_prop_kernel(width):
    per_tile = NCHUNK // (NC * NS)
    return pl.kernel(
        functools.partial(_prop_body, width),
        out_type=jax.ShapeDtypeStruct((NC, N_PAD, width), jnp.float32),
        mesh=_mesh,
        scratch_types=[
            pltpu.VMEM((per_tile // 2, B), jnp.int32),
            pltpu.VMEM((per_tile // 2, B), jnp.int32),
            pltpu.VMEM((2, B, width), jnp.float32),
            pltpu.VMEM_SHARED((N_PAD, width), jnp.float32),
            pltpu.SemaphoreType.DMA,
            pltpu.SemaphoreType.DMA,
            pltpu.SemaphoreType.DMA,
            pltpu.SemaphoreType.DMA,
        ],
    )


_prop128 = _make_prop_kernel(D_H)


def _k2_body(feat_ref, w1_ref, deg_ref, y_ref):
    rs_out = lax.rsqrt(jnp.maximum(deg_ref[0], 1.0))
    y = jnp.dot(feat_ref[...], w1_ref[...], preferred_element_type=jnp.float32)
    y_ref[...] = y * rs_out


def _k4_body(p_ref, deg_ref, b1_ref, w2_ref, z_ref):
    p = p_ref[0] + p_ref[1]
    rs_in = lax.rsqrt(jnp.maximum(deg_ref[1], 1.0))
    rs_out = lax.rsqrt(jnp.maximum(deg_ref[0], 1.0))
    h1 = jnp.maximum(p * rs_in + b1_ref[...], 0.0)
    z = jnp.dot(h1 * rs_out, w2_ref[...], preferred_element_type=jnp.float32)
    z_ref[...] = jnp.concatenate([z, jnp.zeros_like(z)], axis=1)


def _k6_body(p_ref, deg_ref, b2_ref, o_ref):
    p = p_ref[0, :, :D_OUT] + p_ref[1, :, :D_OUT]
    rs_in = lax.rsqrt(jnp.maximum(deg_ref[1], 1.0))
    o_ref[...] = p * rs_in + b2_ref[...]


_GRID = N_PAD // RB

_k2 = pl.pallas_call(
    _k2_body,
    grid=(_GRID,),
    in_specs=[
        pl.BlockSpec((RB, D_IN), lambda i: (i, 0)),
        pl.BlockSpec((D_IN, D_H), lambda i: (0, 0)),
        pl.BlockSpec((NC, RB, 1), lambda i: (0, i, 0)),
    ],
    out_specs=pl.BlockSpec((RB, D_H), lambda i: (i, 0)),
    out_shape=jax.ShapeDtypeStruct((N_PAD, D_H), jnp.float32),
)

_k4 = pl.pallas_call(
    _k4_body,
    grid=(_GRID,),
    in_specs=[
        pl.BlockSpec((NC, RB, D_H), lambda i: (0, i, 0)),
        pl.BlockSpec((NC, RB, 1), lambda i: (0, i, 0)),
        pl.BlockSpec((1, D_H), lambda i: (0, 0)),
        pl.BlockSpec((D_H, D_OUT), lambda i: (0, 0)),
    ],
    out_specs=pl.BlockSpec((RB, D_H), lambda i: (i, 0)),
    out_shape=jax.ShapeDtypeStruct((N_PAD, D_H), jnp.float32),
)

_k6 = pl.pallas_call(
    _k6_body,
    grid=(_GRID,),
    in_specs=[
        pl.BlockSpec((NC, RB, D_H), lambda i: (0, i, 0)),
        pl.BlockSpec((NC, RB, 1), lambda i: (0, i, 0)),
        pl.BlockSpec((1, D_OUT), lambda i: (0, 0)),
    ],
    out_specs=pl.BlockSpec((RB, D_OUT), lambda i: (i, 0)),
    out_shape=jax.ShapeDtypeStruct((N_PAD, D_OUT), jnp.float32),
)


@jax.jit
def kernel(feat, edge_index, W1, b1, W2, b2):
    feat_pad = jnp.zeros((N_PAD, D_IN), jnp.float32).at[:N].set(feat)
    npad = E_PAD - E
    pad_idx = (N + (jnp.arange(npad, dtype=jnp.int32) % (N_PAD - N)))
    ei_pad = jnp.concatenate(
        [edge_index, jnp.broadcast_to(pad_idx, (2, npad))], axis=1
    ).reshape(2, NCHUNK, B)

    zcol = jnp.zeros((N_PAD,), jnp.float32)
    zeros128 = jnp.zeros((N_PAD, D_H), jnp.float32)

    deg = _deg_kernel(ei_pad, zcol)
    deg3 = deg.reshape(NC, N_PAD, 1)

    y = _k2(feat_pad, W1, deg3)
    p1 = _prop128(y, ei_pad, zeros128)
    z = _k4(p1, deg3, b1.reshape(1, D_H), W2)
    p2 = _prop128(z, ei_pad, zeros128)
    out = _k6(p2, deg3, b2.reshape(1, D_OUT))
    return out[:N]


# speedup vs baseline: 12.6965x; 1.0150x over previous
import functools

import jax
import jax.numpy as jnp
from jax import lax
from jax.experimental import pallas as pl
from jax.experimental.pallas import tpu as pltpu
from jax.experimental.pallas import tpu_sc as plsc

N = 10000
E = 320000
D_IN = 128
D_H = 128
D_OUT = 64

NC = 2
NS = 16
B = 128
N_PAD = 10240
ROWS_PER_TILE = N_PAD // NS
E_PAD = 327680
NCHUNK = E_PAD // B
RB = 1024

_mesh = plsc.VectorSubcoreMesh(
    core_axis_name="c", subcore_axis_name="s", num_cores=NC, num_subcores=NS
)


def _deg_body(ei_hbm, zcol_hbm, deg_hbm, idx_all, ones_v, hist_sh, sem_s):
    c = lax.axis_index("c")
    s = lax.axis_index("s")
    pltpu.sync_copy(
        zcol_hbm.at[pl.ds(s * ROWS_PER_TILE, ROWS_PER_TILE)],
        hist_sh.at[pl.ds(s * ROWS_PER_TILE, ROWS_PER_TILE)],
    )
    for k in range(B // 16):
        ones_v[pl.ds(k * 16, 16)] = jnp.full((16,), 1.0, jnp.float32)
    per_tile = NCHUNK // NS
    pltpu.sync_copy(ei_hbm.at[c, pl.ds(s * per_tile, per_tile)], idx_all)
    plsc.subcore_barrier()

    GRP = 16

    @pl.loop(0, per_tile // GRP)
    def _grp(g):
        for j in range(GRP):
            pltpu.async_copy(ones_v, hist_sh.at[idx_all.at[g * GRP + j]],
                             sem_s, add=True)
        for j in range(GRP):
            pltpu.make_async_copy(
                ones_v, hist_sh.at[idx_all.at[g * GRP + j]], sem_s).wait()

    plsc.subcore_barrier()
    pltpu.sync_copy(
        hist_sh.at[pl.ds(s * ROWS_PER_TILE, ROWS_PER_TILE)],
        deg_hbm.at[c, pl.ds(s * ROWS_PER_TILE, ROWS_PER_TILE)],
    )


_deg_kernel = pl.kernel(
    _deg_body,
    out_type=jax.ShapeDtypeStruct((NC, N_PAD), jnp.float32),
    mesh=_mesh,
    scratch_types=[
        pltpu.VMEM((NCHUNK // NS, B), jnp.int32),
        pltpu.VMEM((B,), jnp.float32),
        pltpu.VMEM_SHARED((N_PAD,), jnp.float32),
        pltpu.SemaphoreType.DMA,
    ],
)


def _prop_body(width, nbuf, nstage, tab_hbm, ei_hbm, zer_hbm, out_hbm,
               idx_src, idx_dst, rows_v, acc_sh, sem_g, sem_s):
    c = lax.axis_index("c")
    s = lax.axis_index("s")
    pltpu.sync_copy(
        zer_hbm.at[pl.ds(s * ROWS_PER_TILE, ROWS_PER_TILE), :],
        acc_sh.at[pl.ds(s * ROWS_PER_TILE, ROWS_PER_TILE), :],
    )
    per_tile = NCHUNK // (NC * NS)
    stage = per_tile // nstage
    base = c * (NCHUNK // NC) + s * per_tile

    def gather(t, b):
        pltpu.async_copy(tab_hbm.at[idx_src.at[t]], rows_v.at[b],
                         sem_g.at[b])

    def gather_wait(t, b):
        pltpu.make_async_copy(tab_hbm.at[idx_src.at[t]], rows_v.at[b],
                              sem_g.at[b]).wait()

    def scat(t, b):
        pltpu.async_copy(rows_v.at[b], acc_sh.at[idx_dst.at[t]],
                         sem_s.at[b], add=True)

    def scat_wait(t, b):
        pltpu.make_async_copy(rows_v.at[b], acc_sh.at[idx_dst.at[t]],
                              sem_s.at[b]).wait()

    plsc.subcore_barrier()
    ngrp = stage // nbuf

    @pl.loop(0, nstage)
    def _stage(h):
        pltpu.sync_copy(ei_hbm.at[0, pl.ds(base + h * stage, stage)], idx_src)
        pltpu.sync_copy(ei_hbm.at[1, pl.ds(base + h * stage, stage)], idx_dst)
        for j in range(nbuf):
            gather(j, j)

        @pl.loop(0, ngrp)
        def _grp(i):
            t0 = i * nbuf
            for j in range(nbuf):
                gather_wait(t0 + j, j)
                scat(t0 + j, j)
            for j in range(nbuf):
                scat_wait(t0 + j, j)

                @pl.when(t0 + j + nbuf < stage)
                def _():
                    gather(t0 + j + nbuf, j)

    plsc.subcore_barrier()
    pltpu.sync_copy(
        acc_sh.at[pl.ds(s * ROWS_PER_TILE, ROWS_PER_TILE), :],
        out_hbm.at[c, pl.ds(s * ROWS_PER_TILE, ROWS_PER_TILE), :],
    )


def _make_prop_kernel(width, nbuf, nstage, tc_tiling):
    per_tile = NCHUNK // (NC * NS)
    return pl.kernel(
        functools.partial(_prop_body, width, nbuf, nstage),
        out_type=jax.ShapeDtypeStruct((NC, N_PAD, width), jnp.float32),
        mesh=_mesh,
        scratch_types=[
            pltpu.VMEM((per_tile // nstage, B), jnp.int32),
            pltpu.VMEM((per_tile // nstage, B), jnp.int32),
            pltpu.VMEM((nbuf, B, width), jnp.float32),
            pltpu.VMEM_SHARED((N_PAD, width), jnp.float32),
            pltpu.SemaphoreType.DMA((nbuf,)),
            pltpu.SemaphoreType.DMA((nbuf,)),
        ],
        compiler_params=pltpu.CompilerParams(use_tc_tiling_on_sc=tc_tiling),
    )


_prop128 = _make_prop_kernel(D_H, 2, 2, True)
_prop64 = _make_prop_kernel(D_OUT, 4, 1, False)


def _k2_body(feat_ref, w1_ref, deg_ref, y_ref):
    rs_out = lax.rsqrt(jnp.maximum(deg_ref[0], 1.0))
    y = jnp.dot(feat_ref[...], w1_ref[...], preferred_element_type=jnp.float32)
    y_ref[...] = y * rs_out


def _k4_body(p_ref, deg_ref, b1_ref, w2_ref, z_ref):
    p = p_ref[0] + p_ref[1]
    rs_in = lax.rsqrt(jnp.maximum(deg_ref[1], 1.0))
    rs_out = lax.rsqrt(jnp.maximum(deg_ref[0], 1.0))
    h1 = jnp.maximum(p * rs_in + b1_ref[...], 0.0)
    z_ref[...] = jnp.dot(h1 * rs_out, w2_ref[...],
                         preferred_element_type=jnp.float32)


def _k6_body(p_ref, deg_ref, b2_ref, o_ref):
    p = p_ref[0] + p_ref[1]
    rs_in = lax.rsqrt(jnp.maximum(deg_ref[1], 1.0))
    o_ref[...] = p * rs_in + b2_ref[...]


_GRID = N_PAD // RB

_k2 = pl.pallas_call(
    _k2_body,
    grid=(_GRID,),
    in_specs=[
        pl.BlockSpec((RB, D_IN), lambda i: (i, 0)),
        pl.BlockSpec((D_IN, D_H), lambda i: (0, 0)),
        pl.BlockSpec((NC, RB, 1), lambda i: (0, i, 0)),
    ],
    out_specs=pl.BlockSpec((RB, D_H), lambda i: (i, 0)),
    out_shape=jax.ShapeDtypeStruct((N_PAD, D_H), jnp.float32),
)

_k4 = pl.pallas_call(
    _k4_body,
    grid=(_GRID,),
    in_specs=[
        pl.BlockSpec((NC, RB, D_H), lambda i: (0, i, 0)),
        pl.BlockSpec((NC, RB, 1), lambda i: (0, i, 0)),
        pl.BlockSpec((1, D_H), lambda i: (0, 0)),
        pl.BlockSpec((D_H, D_OUT), lambda i: (0, 0)),
    ],
    out_specs=pl.BlockSpec((RB, D_OUT), lambda i: (i, 0)),
    out_shape=jax.ShapeDtypeStruct((N_PAD, D_OUT), jnp.float32),
)

_k6 = pl.pallas_call(
    _k6_body,
    grid=(_GRID,),
    in_specs=[
        pl.BlockSpec((NC, RB, D_OUT), lambda i: (0, i, 0)),
        pl.BlockSpec((NC, RB, 1), lambda i: (0, i, 0)),
        pl.BlockSpec((1, D_OUT), lambda i: (0, 0)),
    ],
    out_specs=pl.BlockSpec((RB, D_OUT), lambda i: (i, 0)),
    out_shape=jax.ShapeDtypeStruct((N_PAD, D_OUT), jnp.float32),
)


@jax.jit
def kernel(feat, edge_index, W1, b1, W2, b2):
    feat_pad = jnp.zeros((N_PAD, D_IN), jnp.float32).at[:N].set(feat)
    npad = E_PAD - E
    pad_idx = (N + (jnp.arange(npad, dtype=jnp.int32) % (N_PAD - N)))
    ei_pad = jnp.concatenate(
        [edge_index, jnp.broadcast_to(pad_idx, (2, npad))], axis=1
    ).reshape(2, NCHUNK, B)

    zcol = jnp.zeros((N_PAD,), jnp.float32)
    zeros128 = jnp.zeros((N_PAD, D_H), jnp.float32)
    zeros64 = jnp.zeros((N_PAD, D_OUT), jnp.float32)

    deg = _deg_kernel(ei_pad, zcol)
    deg3 = deg.reshape(NC, N_PAD, 1)

    y = _k2(feat_pad, W1, deg3)
    p1 = _prop128(y, ei_pad, zeros128)
    z = _k4(p1, deg3, b1.reshape(1, D_H), W2)
    p2 = _prop64(z, ei_pad, zeros64)
    out = _k6(p2, deg3, b2.reshape(1, D_OUT))
    return out[:N]


# speedup vs baseline: 13.7534x; 1.0832x over previous
import functools

import jax
import jax.numpy as jnp
from jax import lax
from jax.experimental import pallas as pl
from jax.experimental.pallas import tpu as pltpu
from jax.experimental.pallas import tpu_sc as plsc

N = 10000
E = 320000
D_IN = 128
D_H = 128
D_OUT = 64

NC = 2
NS = 16
B = 128
N_PAD = 10240
ROWS_PER_TILE = N_PAD // NS
E_PAD = 327680
NCHUNK = E_PAD // B
RB = 1024

_mesh = plsc.VectorSubcoreMesh(
    core_axis_name="c", subcore_axis_name="s", num_cores=NC, num_subcores=NS
)


def _deg_body(ei_hbm, zcol_hbm, deg_hbm, idx_all, ones_v, hist_sh, sem_s):
    c = lax.axis_index("c")
    s = lax.axis_index("s")
    pltpu.sync_copy(
        zcol_hbm.at[pl.ds(s * ROWS_PER_TILE, ROWS_PER_TILE)],
        hist_sh.at[pl.ds(s * ROWS_PER_TILE, ROWS_PER_TILE)],
    )
    for k in range(B // 16):
        ones_v[pl.ds(k * 16, 16)] = jnp.full((16,), 1.0, jnp.float32)
    per_tile = NCHUNK // NS
    pltpu.sync_copy(ei_hbm.at[c, pl.ds(s * per_tile, per_tile)], idx_all)
    plsc.subcore_barrier()

    GRP = 16

    @pl.loop(0, per_tile // GRP)
    def _grp(g):
        for j in range(GRP):
            pltpu.async_copy(ones_v, hist_sh.at[idx_all.at[g * GRP + j]],
                             sem_s, add=True)
        for j in range(GRP):
            pltpu.make_async_copy(
                ones_v, hist_sh.at[idx_all.at[g * GRP + j]], sem_s).wait()

    plsc.subcore_barrier()
    pltpu.sync_copy(
        hist_sh.at[pl.ds(s * ROWS_PER_TILE, ROWS_PER_TILE)],
        deg_hbm.at[c, pl.ds(s * ROWS_PER_TILE, ROWS_PER_TILE)],
    )


_deg_kernel = pl.kernel(
    _deg_body,
    out_type=jax.ShapeDtypeStruct((NC, N_PAD), jnp.float32),
    mesh=_mesh,
    scratch_types=[
        pltpu.VMEM((NCHUNK // NS, B), jnp.int32),
        pltpu.VMEM((B,), jnp.float32),
        pltpu.VMEM_SHARED((N_PAD,), jnp.float32),
        pltpu.SemaphoreType.DMA,
    ],
)


def _prop_body(width, nbuf, nstage, tab_hbm, ei_hbm, zer_hbm, out_hbm,
               idx_src, idx_dst, rows_v, acc_sh, sem_g, sem_s):
    c = lax.axis_index("c")
    s = lax.axis_index("s")
    pltpu.sync_copy(
        zer_hbm.at[pl.ds(s * ROWS_PER_TILE, ROWS_PER_TILE), :],
        acc_sh.at[pl.ds(s * ROWS_PER_TILE, ROWS_PER_TILE), :],
    )
    per_tile = NCHUNK // (NC * NS)
    stage = per_tile // nstage
    base = c * (NCHUNK // NC) + s * per_tile

    def gather(t, b):
        pltpu.async_copy(tab_hbm.at[idx_src.at[t]], rows_v.at[b],
                         sem_g.at[b])

    def gather_wait(t, b):
        pltpu.make_async_copy(tab_hbm.at[idx_src.at[t]], rows_v.at[b],
                              sem_g.at[b]).wait()

    def scat(t, b):
        pltpu.async_copy(rows_v.at[b], acc_sh.at[idx_dst.at[t]],
                         sem_s.at[b], add=True)

    def scat_wait(t, b):
        pltpu.make_async_copy(rows_v.at[b], acc_sh.at[idx_dst.at[t]],
                              sem_s.at[b]).wait()

    plsc.subcore_barrier()
    ngrp = stage // nbuf

    @pl.loop(0, nstage)
    def _stage(h):
        pltpu.sync_copy(ei_hbm.at[0, pl.ds(base + h * stage, stage)], idx_src)
        pltpu.sync_copy(ei_hbm.at[1, pl.ds(base + h * stage, stage)], idx_dst)
        for j in range(nbuf):
            gather(j, j)

        @pl.loop(0, ngrp)
        def _grp(i):
            t0 = i * nbuf
            for j in range(nbuf):
                t = t0 + j
                jp = (j - 1) % nbuf
                gather_wait(t, j)

                @pl.when(t > 0)
                def _():
                    scat_wait(t - 1, jp)

                scat(t, j)

                @pl.when((t > 0) & (t - 1 + nbuf < stage))
                def _():
                    gather(t - 1 + nbuf, jp)

        scat_wait(stage - 1, (stage - 1) % nbuf)

    plsc.subcore_barrier()
    pltpu.sync_copy(
        acc_sh.at[pl.ds(s * ROWS_PER_TILE, ROWS_PER_TILE), :],
        out_hbm.at[c, pl.ds(s * ROWS_PER_TILE, ROWS_PER_TILE), :],
    )


def _make_prop_kernel(width, nbuf, nstage, tc_tiling):
    per_tile = NCHUNK // (NC * NS)
    return pl.kernel(
        functools.partial(_prop_body, width, nbuf, nstage),
        out_type=jax.ShapeDtypeStruct((NC, N_PAD, width), jnp.float32),
        mesh=_mesh,
        scratch_types=[
            pltpu.VMEM((per_tile // nstage, B), jnp.int32),
            pltpu.VMEM((per_tile // nstage, B), jnp.int32),
            pltpu.VMEM((nbuf, B, width), jnp.float32),
            pltpu.VMEM_SHARED((N_PAD, width), jnp.float32),
            pltpu.SemaphoreType.DMA((nbuf,)),
            pltpu.SemaphoreType.DMA((nbuf,)),
        ],
        compiler_params=pltpu.CompilerParams(use_tc_tiling_on_sc=tc_tiling),
    )


_prop128 = _make_prop_kernel(D_H, 2, 2, True)
_prop64 = _make_prop_kernel(D_OUT, 4, 1, False)


def _k2_body(feat_ref, w1_ref, deg_ref, y_ref):
    rs_out = lax.rsqrt(jnp.maximum(deg_ref[0], 1.0))
    y = jnp.dot(feat_ref[...], w1_ref[...], preferred_element_type=jnp.float32)
    y_ref[...] = y * rs_out


def _k4_body(p_ref, deg_ref, b1_ref, w2_ref, z_ref):
    p = p_ref[0] + p_ref[1]
    rs_in = lax.rsqrt(jnp.maximum(deg_ref[1], 1.0))
    rs_out = lax.rsqrt(jnp.maximum(deg_ref[0], 1.0))
    h1 = jnp.maximum(p * rs_in + b1_ref[...], 0.0)
    z_ref[...] = jnp.dot(h1 * rs_out, w2_ref[...],
                         preferred_element_type=jnp.float32)


def _k6_body(p_ref, deg_ref, b2_ref, o_ref):
    p = p_ref[0] + p_ref[1]
    rs_in = lax.rsqrt(jnp.maximum(deg_ref[1], 1.0))
    o_ref[...] = p * rs_in + b2_ref[...]


_GRID = N_PAD // RB

_k2 = pl.pallas_call(
    _k2_body,
    grid=(_GRID,),
    in_specs=[
        pl.BlockSpec((RB, D_IN), lambda i: (i, 0)),
        pl.BlockSpec((D_IN, D_H), lambda i: (0, 0)),
        pl.BlockSpec((NC, RB, 1), lambda i: (0, i, 0)),
    ],
    out_specs=pl.BlockSpec((RB, D_H), lambda i: (i, 0)),
    out_shape=jax.ShapeDtypeStruct((N_PAD, D_H), jnp.float32),
)

_k4 = pl.pallas_call(
    _k4_body,
    grid=(_GRID,),
    in_specs=[
        pl.BlockSpec((NC, RB, D_H), lambda i: (0, i, 0)),
        pl.BlockSpec((NC, RB, 1), lambda i: (0, i, 0)),
        pl.BlockSpec((1, D_H), lambda i: (0, 0)),
        pl.BlockSpec((D_H, D_OUT), lambda i: (0, 0)),
    ],
    out_specs=pl.BlockSpec((RB, D_OUT), lambda i: (i, 0)),
    out_shape=jax.ShapeDtypeStruct((N_PAD, D_OUT), jnp.float32),
)

_k6 = pl.pallas_call(
    _k6_body,
    grid=(_GRID,),
    in_specs=[
        pl.BlockSpec((NC, RB, D_OUT), lambda i: (0, i, 0)),
        pl.BlockSpec((NC, RB, 1), lambda i: (0, i, 0)),
        pl.BlockSpec((1, D_OUT), lambda i: (0, 0)),
    ],
    out_specs=pl.BlockSpec((RB, D_OUT), lambda i: (i, 0)),
    out_shape=jax.ShapeDtypeStruct((N_PAD, D_OUT), jnp.float32),
)


@jax.jit
def kernel(feat, edge_index, W1, b1, W2, b2):
    feat_pad = jnp.zeros((N_PAD, D_IN), jnp.float32).at[:N].set(feat)
    npad = E_PAD - E
    pad_idx = (N + (jnp.arange(npad, dtype=jnp.int32) % (N_PAD - N)))
    ei_pad = jnp.concatenate(
        [edge_index, jnp.broadcast_to(pad_idx, (2, npad))], axis=1
    ).reshape(2, NCHUNK, B)

    zcol = jnp.zeros((N_PAD,), jnp.float32)
    zeros128 = jnp.zeros((N_PAD, D_H), jnp.float32)
    zeros64 = jnp.zeros((N_PAD, D_OUT), jnp.float32)

    deg = _deg_kernel(ei_pad, zcol)
    deg3 = deg.reshape(NC, N_PAD, 1)

    y = _k2(feat_pad, W1, deg3)
    p1 = _prop128(y, ei_pad, zeros128)
    z = _k4(p1, deg3, b1.reshape(1, D_H), W2)
    p2 = _prop64(z, ei_pad, zeros64)
    out = _k6(p2, deg3, b2.reshape(1, D_OUT))
    return out[:N]


# speedup vs baseline: 13.7569x; 1.0003x over previous
import functools

import jax
import jax.numpy as jnp
from jax import lax
from jax.experimental import pallas as pl
from jax.experimental.pallas import tpu as pltpu
from jax.experimental.pallas import tpu_sc as plsc

N = 10000
E = 320000
D_IN = 128
D_H = 128
D_OUT = 64

NC = 2
NS = 16
B = 128
N_PAD = 10240
ROWS_PER_TILE = N_PAD // NS
E_PAD = 327680
NCHUNK = E_PAD // B
RB = 1024

_mesh = plsc.VectorSubcoreMesh(
    core_axis_name="c", subcore_axis_name="s", num_cores=NC, num_subcores=NS
)


def _deg_body(ei_hbm, zcol_hbm, deg_hbm, idx_all, ones_v, hist_sh, sem_s):
    c = lax.axis_index("c")
    s = lax.axis_index("s")
    pltpu.sync_copy(
        zcol_hbm.at[pl.ds(s * ROWS_PER_TILE, ROWS_PER_TILE)],
        hist_sh.at[pl.ds(s * ROWS_PER_TILE, ROWS_PER_TILE)],
    )
    for k in range(B // 16):
        ones_v[pl.ds(k * 16, 16)] = jnp.full((16,), 1.0, jnp.float32)
    per_tile = NCHUNK // NS
    pltpu.sync_copy(ei_hbm.at[c, pl.ds(s * per_tile, per_tile)], idx_all)
    plsc.subcore_barrier()

    GRP = 16

    @pl.loop(0, per_tile // GRP)
    def _grp(g):
        for j in range(GRP):
            pltpu.async_copy(ones_v, hist_sh.at[idx_all.at[g * GRP + j]],
                             sem_s, add=True)
        for j in range(GRP):
            pltpu.make_async_copy(
                ones_v, hist_sh.at[idx_all.at[g * GRP + j]], sem_s).wait()

    plsc.subcore_barrier()
    pltpu.sync_copy(
        hist_sh.at[pl.ds(s * ROWS_PER_TILE, ROWS_PER_TILE)],
        deg_hbm.at[c, pl.ds(s * ROWS_PER_TILE, ROWS_PER_TILE)],
    )


_deg_kernel = pl.kernel(
    _deg_body,
    out_type=jax.ShapeDtypeStruct((NC, N_PAD), jnp.float32),
    mesh=_mesh,
    scratch_types=[
        pltpu.VMEM((NCHUNK // NS, B), jnp.int32),
        pltpu.VMEM((B,), jnp.float32),
        pltpu.VMEM_SHARED((N_PAD,), jnp.float32),
        pltpu.SemaphoreType.DMA,
    ],
)


def _prop_body(width, nbuf, nstage, tab_hbm, ei_hbm, zer_hbm, out_hbm,
               idx_src, idx_dst, rows_v, acc_sh, sem_g, sem_s):
    c = lax.axis_index("c")
    s = lax.axis_index("s")
    pltpu.sync_copy(
        zer_hbm.at[pl.ds(s * ROWS_PER_TILE, ROWS_PER_TILE), :],
        acc_sh.at[pl.ds(s * ROWS_PER_TILE, ROWS_PER_TILE), :],
    )
    per_tile = NCHUNK // (NC * NS)
    stage = per_tile // nstage
    base = c * (NCHUNK // NC) + s * per_tile

    def gather(t, b):
        pltpu.async_copy(tab_hbm.at[idx_src.at[t]], rows_v.at[b],
                         sem_g.at[b])

    def gather_wait(t, b):
        pltpu.make_async_copy(tab_hbm.at[idx_src.at[t]], rows_v.at[b],
                              sem_g.at[b]).wait()

    def scat(t, b):
        pltpu.async_copy(rows_v.at[b], acc_sh.at[idx_dst.at[t]],
                         sem_s.at[b], add=True)

    def scat_wait(t, b):
        pltpu.make_async_copy(rows_v.at[b], acc_sh.at[idx_dst.at[t]],
                              sem_s.at[b]).wait()

    plsc.subcore_barrier()
    ngrp = stage // nbuf

    @pl.loop(0, nstage)
    def _stage(h):
        pltpu.sync_copy(ei_hbm.at[0, pl.ds(base + h * stage, stage)], idx_src)
        pltpu.sync_copy(ei_hbm.at[1, pl.ds(base + h * stage, stage)], idx_dst)
        for j in range(nbuf):
            gather(j, j)

        @pl.loop(0, ngrp)
        def _grp(i):
            t0 = i * nbuf
            for j in range(nbuf):
                t = t0 + j
                jp = (j - 1) % nbuf
                gather_wait(t, j)

                @pl.when(t > 0)
                def _():
                    scat_wait(t - 1, jp)

                scat(t, j)

                @pl.when((t > 0) & (t - 1 + nbuf < stage))
                def _():
                    gather(t - 1 + nbuf, jp)

        scat_wait(stage - 1, (stage - 1) % nbuf)

    plsc.subcore_barrier()
    pltpu.sync_copy(
        acc_sh.at[pl.ds(s * ROWS_PER_TILE, ROWS_PER_TILE), :],
        out_hbm.at[c, pl.ds(s * ROWS_PER_TILE, ROWS_PER_TILE), :],
    )


def _make_prop_kernel(width, nbuf, nstage, tc_tiling):
    per_tile = NCHUNK // (NC * NS)
    return pl.kernel(
        functools.partial(_prop_body, width, nbuf, nstage),
        out_type=jax.ShapeDtypeStruct((NC, N_PAD, width), jnp.float32),
        mesh=_mesh,
        scratch_types=[
            pltpu.VMEM((per_tile // nstage, B), jnp.int32),
            pltpu.VMEM((per_tile // nstage, B), jnp.int32),
            pltpu.VMEM((nbuf, B, width), jnp.float32),
            pltpu.VMEM_SHARED((N_PAD, width), jnp.float32),
            pltpu.SemaphoreType.DMA((nbuf,)),
            pltpu.SemaphoreType.DMA((nbuf,)),
        ],
        compiler_params=pltpu.CompilerParams(use_tc_tiling_on_sc=tc_tiling),
    )


_prop128 = _make_prop_kernel(D_H, 2, 2, False)
_prop64 = _make_prop_kernel(D_OUT, 4, 1, False)


def _k2_body(feat_ref, w1_ref, deg_ref, y_ref):
    rs_out = lax.rsqrt(jnp.maximum(deg_ref[0], 1.0))
    y = jnp.dot(feat_ref[...], w1_ref[...], preferred_element_type=jnp.float32)
    y_ref[...] = y * rs_out


def _k4_body(p_ref, deg_ref, b1_ref, w2_ref, z_ref):
    p = p_ref[0] + p_ref[1]
    rs_in = lax.rsqrt(jnp.maximum(deg_ref[1], 1.0))
    rs_out = lax.rsqrt(jnp.maximum(deg_ref[0], 1.0))
    h1 = jnp.maximum(p * rs_in + b1_ref[...], 0.0)
    z_ref[...] = jnp.dot(h1 * rs_out, w2_ref[...],
                         preferred_element_type=jnp.float32)


def _k6_body(p_ref, deg_ref, b2_ref, o_ref):
    p = p_ref[0] + p_ref[1]
    rs_in = lax.rsqrt(jnp.maximum(deg_ref[1], 1.0))
    o_ref[...] = p * rs_in + b2_ref[...]


_GRID = N_PAD // RB

_k2 = pl.pallas_call(
    _k2_body,
    grid=(_GRID,),
    in_specs=[
        pl.BlockSpec((RB, D_IN), lambda i: (i, 0)),
        pl.BlockSpec((D_IN, D_H), lambda i: (0, 0)),
        pl.BlockSpec((NC, RB, 1), lambda i: (0, i, 0)),
    ],
    out_specs=pl.BlockSpec((RB, D_H), lambda i: (i, 0)),
    out_shape=jax.ShapeDtypeStruct((N_PAD, D_H), jnp.float32),
)

_k4 = pl.pallas_call(
    _k4_body,
    grid=(_GRID,),
    in_specs=[
        pl.BlockSpec((NC, RB, D_H), lambda i: (0, i, 0)),
        pl.BlockSpec((NC, RB, 1), lambda i: (0, i, 0)),
        pl.BlockSpec((1, D_H), lambda i: (0, 0)),
        pl.BlockSpec((D_H, D_OUT), lambda i: (0, 0)),
    ],
    out_specs=pl.BlockSpec((RB, D_OUT), lambda i: (i, 0)),
    out_shape=jax.ShapeDtypeStruct((N_PAD, D_OUT), jnp.float32),
)

_k6 = pl.pallas_call(
    _k6_body,
    grid=(_GRID,),
    in_specs=[
        pl.BlockSpec((NC, RB, D_OUT), lambda i: (0, i, 0)),
        pl.BlockSpec((NC, RB, 1), lambda i: (0, i, 0)),
        pl.BlockSpec((1, D_OUT), lambda i: (0, 0)),
    ],
    out_specs=pl.BlockSpec((RB, D_OUT), lambda i: (i, 0)),
    out_shape=jax.ShapeDtypeStruct((N_PAD, D_OUT), jnp.float32),
)


@jax.jit
def kernel(feat, edge_index, W1, b1, W2, b2):
    feat_pad = jnp.zeros((N_PAD, D_IN), jnp.float32).at[:N].set(feat)
    npad = E_PAD - E
    pad_idx = (N + (jnp.arange(npad, dtype=jnp.int32) % (N_PAD - N)))
    ei_pad = jnp.concatenate(
        [edge_index, jnp.broadcast_to(pad_idx, (2, npad))], axis=1
    ).reshape(2, NCHUNK, B)

    zcol = jnp.zeros((N_PAD,), jnp.float32)
    zeros128 = jnp.zeros((N_PAD, D_H), jnp.float32)
    zeros64 = jnp.zeros((N_PAD, D_OUT), jnp.float32)

    deg = _deg_kernel(ei_pad, zcol)
    deg3 = deg.reshape(NC, N_PAD, 1)

    y = _k2(feat_pad, W1, deg3)
    p1 = _prop128(y, ei_pad, zeros128)
    z = _k4(p1, deg3, b1.reshape(1, D_H), W2)
    p2 = _prop64(z, ei_pad, zeros64)
    out = _k6(p2, deg3, b2.reshape(1, D_OUT))
    return out[:N]


# speedup vs baseline: 15.2815x; 1.1108x over previous
import functools

import jax
import jax.numpy as jnp
from jax import lax
from jax.experimental import pallas as pl
from jax.experimental.pallas import tpu as pltpu
from jax.experimental.pallas import tpu_sc as plsc

N = 10000
E = 320000
D_IN = 128
D_H = 128
D_OUT = 64

NC = 2
NS = 16
B = 128
N_PAD = 10240
ROWS_PER_TILE = N_PAD // NS
E_PAD = 327680
NCHUNK = E_PAD // B
RB = 1024

_mesh = plsc.VectorSubcoreMesh(
    core_axis_name="c", subcore_axis_name="s", num_cores=NC, num_subcores=NS
)


def _deg_body(ei_hbm, zcol_hbm, deg_hbm, idx_all, ones_v, hist_sh, sem_s):
    c = lax.axis_index("c")
    s = lax.axis_index("s")
    pltpu.sync_copy(
        zcol_hbm.at[pl.ds(s * ROWS_PER_TILE, ROWS_PER_TILE)],
        hist_sh.at[pl.ds(s * ROWS_PER_TILE, ROWS_PER_TILE)],
    )
    for k in range(B // 16):
        ones_v[pl.ds(k * 16, 16)] = jnp.full((16,), 1.0, jnp.float32)
    per_tile = NCHUNK // NS
    pltpu.sync_copy(ei_hbm.at[c, pl.ds(s * per_tile, per_tile)], idx_all)
    plsc.subcore_barrier()

    GRP = 16

    @pl.loop(0, per_tile // GRP)
    def _grp(g):
        for j in range(GRP):
            pltpu.async_copy(ones_v, hist_sh.at[idx_all.at[g * GRP + j]],
                             sem_s, add=True)
        for j in range(GRP):
            pltpu.make_async_copy(
                ones_v, hist_sh.at[idx_all.at[g * GRP + j]], sem_s).wait()

    plsc.subcore_barrier()
    pltpu.sync_copy(
        hist_sh.at[pl.ds(s * ROWS_PER_TILE, ROWS_PER_TILE)],
        deg_hbm.at[c, pl.ds(s * ROWS_PER_TILE, ROWS_PER_TILE)],
    )


_deg_kernel = pl.kernel(
    _deg_body,
    out_type=jax.ShapeDtypeStruct((NC, N_PAD), jnp.float32),
    mesh=_mesh,
    scratch_types=[
        pltpu.VMEM((NCHUNK // NS, B), jnp.int32),
        pltpu.VMEM((B,), jnp.float32),
        pltpu.VMEM_SHARED((N_PAD,), jnp.float32),
        pltpu.SemaphoreType.DMA,
    ],
)


def _prop_body(width, nbuf, nstage, cb, tab_hbm, ei_hbm, zer_hbm, out_hbm,
               idx_src, idx_dst, rows_v, acc_sh, sem_g, sem_s):
    c = lax.axis_index("c")
    s = lax.axis_index("s")
    pltpu.sync_copy(
        zer_hbm.at[pl.ds(s * ROWS_PER_TILE, ROWS_PER_TILE), :],
        acc_sh.at[pl.ds(s * ROWS_PER_TILE, ROWS_PER_TILE), :],
    )
    nchunk = E_PAD // cb
    per_tile = nchunk // (NC * NS)
    stage = per_tile // nstage
    base = c * (nchunk // NC) + s * per_tile

    def gather(t, b):
        pltpu.async_copy(tab_hbm.at[idx_src.at[t]], rows_v.at[b],
                         sem_g.at[b])

    def gather_wait(t, b):
        pltpu.make_async_copy(tab_hbm.at[idx_src.at[t]], rows_v.at[b],
                              sem_g.at[b]).wait()

    def scat(t, b):
        pltpu.async_copy(rows_v.at[b], acc_sh.at[idx_dst.at[t]],
                         sem_s.at[b], add=True)

    def scat_wait(t, b):
        pltpu.make_async_copy(rows_v.at[b], acc_sh.at[idx_dst.at[t]],
                              sem_s.at[b]).wait()

    plsc.subcore_barrier()
    ngrp = stage // nbuf

    @pl.loop(0, nstage)
    def _stage(h):
        pltpu.sync_copy(ei_hbm.at[0, pl.ds(base + h * stage, stage)], idx_src)
        pltpu.sync_copy(ei_hbm.at[1, pl.ds(base + h * stage, stage)], idx_dst)
        for j in range(nbuf):
            gather(j, j)

        @pl.loop(0, ngrp)
        def _grp(i):
            t0 = i * nbuf
            for j in range(nbuf):
                t = t0 + j
                jp = (j - 1) % nbuf
                gather_wait(t, j)

                @pl.when(t > 0)
                def _():
                    scat_wait(t - 1, jp)

                scat(t, j)

                @pl.when((t > 0) & (t - 1 + nbuf < stage))
                def _():
                    gather(t - 1 + nbuf, jp)

        scat_wait(stage - 1, (stage - 1) % nbuf)

    plsc.subcore_barrier()
    pltpu.sync_copy(
        acc_sh.at[pl.ds(s * ROWS_PER_TILE, ROWS_PER_TILE), :],
        out_hbm.at[c, pl.ds(s * ROWS_PER_TILE, ROWS_PER_TILE), :],
    )


def _make_prop_kernel(width, nbuf, nstage, cb):
    per_tile = (E_PAD // cb) // (NC * NS)
    return pl.kernel(
        functools.partial(_prop_body, width, nbuf, nstage, cb),
        out_type=jax.ShapeDtypeStruct((NC, N_PAD, width), jnp.float32),
        mesh=_mesh,
        scratch_types=[
            pltpu.VMEM((per_tile // nstage, cb), jnp.int32),
            pltpu.VMEM((per_tile // nstage, cb), jnp.int32),
            pltpu.VMEM((nbuf, cb, width), jnp.float32),
            pltpu.VMEM_SHARED((N_PAD, width), jnp.float32),
            pltpu.SemaphoreType.DMA((nbuf,)),
            pltpu.SemaphoreType.DMA((nbuf,)),
        ],
        compiler_params=pltpu.CompilerParams(use_tc_tiling_on_sc=False),
    )


_prop128 = _make_prop_kernel(D_H, 4, 4, 64)
_prop64 = _make_prop_kernel(D_OUT, 4, 1, B)


def _k2_body(feat_ref, w1_ref, deg_ref, y_ref):
    rs_out = lax.rsqrt(jnp.maximum(deg_ref[0], 1.0))
    y = jnp.dot(feat_ref[...], w1_ref[...], preferred_element_type=jnp.float32)
    y_ref[...] = y * rs_out


def _k4_body(p_ref, deg_ref, b1_ref, w2_ref, z_ref):
    p = p_ref[0] + p_ref[1]
    rs_in = lax.rsqrt(jnp.maximum(deg_ref[1], 1.0))
    rs_out = lax.rsqrt(jnp.maximum(deg_ref[0], 1.0))
    h1 = jnp.maximum(p * rs_in + b1_ref[...], 0.0)
    z_ref[...] = jnp.dot(h1 * rs_out, w2_ref[...],
                         preferred_element_type=jnp.float32)


def _k6_body(p_ref, deg_ref, b2_ref, o_ref):
    p = p_ref[0] + p_ref[1]
    rs_in = lax.rsqrt(jnp.maximum(deg_ref[1], 1.0))
    o_ref[...] = p * rs_in + b2_ref[...]


_GRID = N_PAD // RB

_k2 = pl.pallas_call(
    _k2_body,
    grid=(_GRID,),
    in_specs=[
        pl.BlockSpec((RB, D_IN), lambda i: (i, 0)),
        pl.BlockSpec((D_IN, D_H), lambda i: (0, 0)),
        pl.BlockSpec((NC, RB, 1), lambda i: (0, i, 0)),
    ],
    out_specs=pl.BlockSpec((RB, D_H), lambda i: (i, 0)),
    out_shape=jax.ShapeDtypeStruct((N_PAD, D_H), jnp.float32),
)

_k4 = pl.pallas_call(
    _k4_body,
    grid=(_GRID,),
    in_specs=[
        pl.BlockSpec((NC, RB, D_H), lambda i: (0, i, 0)),
        pl.BlockSpec((NC, RB, 1), lambda i: (0, i, 0)),
        pl.BlockSpec((1, D_H), lambda i: (0, 0)),
        pl.BlockSpec((D_H, D_OUT), lambda i: (0, 0)),
    ],
    out_specs=pl.BlockSpec((RB, D_OUT), lambda i: (i, 0)),
    out_shape=jax.ShapeDtypeStruct((N_PAD, D_OUT), jnp.float32),
)

_k6 = pl.pallas_call(
    _k6_body,
    grid=(_GRID,),
    in_specs=[
        pl.BlockSpec((NC, RB, D_OUT), lambda i: (0, i, 0)),
        pl.BlockSpec((NC, RB, 1), lambda i: (0, i, 0)),
        pl.BlockSpec((1, D_OUT), lambda i: (0, 0)),
    ],
    out_specs=pl.BlockSpec((RB, D_OUT), lambda i: (i, 0)),
    out_shape=jax.ShapeDtypeStruct((N_PAD, D_OUT), jnp.float32),
)


@jax.jit
def kernel(feat, edge_index, W1, b1, W2, b2):
    feat_pad = jnp.zeros((N_PAD, D_IN), jnp.float32).at[:N].set(feat)
    npad = E_PAD - E
    pad_idx = (N + (jnp.arange(npad, dtype=jnp.int32) % (N_PAD - N)))
    ei_pad = jnp.concatenate(
        [edge_index, jnp.broadcast_to(pad_idx, (2, npad))], axis=1
    ).reshape(2, NCHUNK, B)

    zcol = jnp.zeros((N_PAD,), jnp.float32)
    zeros128 = jnp.zeros((N_PAD, D_H), jnp.float32)
    zeros64 = jnp.zeros((N_PAD, D_OUT), jnp.float32)

    deg = _deg_kernel(ei_pad, zcol)
    deg3 = deg.reshape(NC, N_PAD, 1)

    ei64 = ei_pad.reshape(2, E_PAD // 64, 64)

    y = _k2(feat_pad, W1, deg3)
    p1 = _prop128(y, ei64, zeros128)
    z = _k4(p1, deg3, b1.reshape(1, D_H), W2)
    p2 = _prop64(z, ei_pad, zeros64)
    out = _k6(p2, deg3, b2.reshape(1, D_OUT))
    return out[:N]


# speedup vs baseline: 15.9206x; 1.0418x over previous
import functools

import jax
import jax.numpy as jnp
from jax import lax
from jax.experimental import pallas as pl
from jax.experimental.pallas import tpu as pltpu
from jax.experimental.pallas import tpu_sc as plsc

N = 10000
E = 320000
D_IN = 128
D_H = 128
D_OUT = 64

NC = 2
NS = 16
B = 128
N_PAD = 10240
ROWS_PER_TILE = N_PAD // NS
E_PAD = 327680
NCHUNK = E_PAD // B
RB = 1024

_mesh = plsc.VectorSubcoreMesh(
    core_axis_name="c", subcore_axis_name="s", num_cores=NC, num_subcores=NS
)


def _deg_body(ei_hbm, zcol_hbm, deg_hbm, idx_all, ones_v, hist_sh, sem_s):
    c = lax.axis_index("c")
    s = lax.axis_index("s")
    pltpu.sync_copy(
        zcol_hbm.at[pl.ds(s * ROWS_PER_TILE, ROWS_PER_TILE)],
        hist_sh.at[pl.ds(s * ROWS_PER_TILE, ROWS_PER_TILE)],
    )
    for k in range(B // 16):
        ones_v[pl.ds(k * 16, 16)] = jnp.full((16,), 1.0, jnp.float32)
    per_tile = NCHUNK // NS
    pltpu.sync_copy(ei_hbm.at[c, pl.ds(s * per_tile, per_tile)], idx_all)
    plsc.subcore_barrier()

    GRP = 16

    @pl.loop(0, per_tile // GRP)
    def _grp(g):
        for j in range(GRP):
            pltpu.async_copy(ones_v, hist_sh.at[idx_all.at[g * GRP + j]],
                             sem_s, add=True)
        for j in range(GRP):
            pltpu.make_async_copy(
                ones_v, hist_sh.at[idx_all.at[g * GRP + j]], sem_s).wait()

    plsc.subcore_barrier()
    pltpu.sync_copy(
        hist_sh.at[pl.ds(s * ROWS_PER_TILE, ROWS_PER_TILE)],
        deg_hbm.at[c, pl.ds(s * ROWS_PER_TILE, ROWS_PER_TILE)],
    )


_deg_kernel = pl.kernel(
    _deg_body,
    out_type=jax.ShapeDtypeStruct((NC, N_PAD), jnp.float32),
    mesh=_mesh,
    scratch_types=[
        pltpu.VMEM((NCHUNK // NS, B), jnp.int32),
        pltpu.VMEM((B,), jnp.float32),
        pltpu.VMEM_SHARED((N_PAD,), jnp.float32),
        pltpu.SemaphoreType.DMA,
    ],
)


def _prop_body(width, nbuf, nstage, cb, tab_hbm, ei_hbm, zer_hbm, out_hbm,
               idx_src, idx_dst, rows_v, acc_sh, sem_g, sem_s):
    c = lax.axis_index("c")
    s = lax.axis_index("s")
    pltpu.sync_copy(
        zer_hbm.at[pl.ds(s * ROWS_PER_TILE, ROWS_PER_TILE), :],
        acc_sh.at[pl.ds(s * ROWS_PER_TILE, ROWS_PER_TILE), :],
    )
    nchunk = E_PAD // cb
    per_tile = nchunk // (NC * NS)
    stage = per_tile // nstage
    base = c * (nchunk // NC) + s * per_tile

    def gather(t, b):
        pltpu.async_copy(tab_hbm.at[idx_src.at[t]], rows_v.at[b],
                         sem_g.at[b])

    def gather_wait(t, b):
        pltpu.make_async_copy(tab_hbm.at[idx_src.at[t]], rows_v.at[b],
                              sem_g.at[b]).wait()

    def scat(t, b):
        pltpu.async_copy(rows_v.at[b], acc_sh.at[idx_dst.at[t]],
                         sem_s.at[b], add=True)

    def scat_wait(t, b):
        pltpu.make_async_copy(rows_v.at[b], acc_sh.at[idx_dst.at[t]],
                              sem_s.at[b]).wait()

    plsc.subcore_barrier()
    ngrp = stage // nbuf

    @pl.loop(0, nstage)
    def _stage(h):
        pltpu.sync_copy(ei_hbm.at[0, pl.ds(base + h * stage, stage)], idx_src)
        pltpu.sync_copy(ei_hbm.at[1, pl.ds(base + h * stage, stage)], idx_dst)
        for j in range(nbuf):
            gather(j, j)

        @pl.loop(0, ngrp)
        def _grp(i):
            t0 = i * nbuf
            for j in range(nbuf):
                t = t0 + j
                jp = (j - 1) % nbuf
                gather_wait(t, j)

                @pl.when(t > 0)
                def _():
                    scat_wait(t - 1, jp)

                scat(t, j)

                @pl.when((t > 0) & (t - 1 + nbuf < stage))
                def _():
                    gather(t - 1 + nbuf, jp)

        scat_wait(stage - 1, (stage - 1) % nbuf)

    plsc.subcore_barrier()
    pltpu.sync_copy(
        acc_sh.at[pl.ds(s * ROWS_PER_TILE, ROWS_PER_TILE), :],
        out_hbm.at[c, pl.ds(s * ROWS_PER_TILE, ROWS_PER_TILE), :],
    )


def _make_prop_kernel(width, nbuf, nstage, cb):
    per_tile = (E_PAD // cb) // (NC * NS)
    return pl.kernel(
        functools.partial(_prop_body, width, nbuf, nstage, cb),
        out_type=jax.ShapeDtypeStruct((NC, N_PAD, width), jnp.float32),
        mesh=_mesh,
        scratch_types=[
            pltpu.VMEM((per_tile // nstage, cb), jnp.int32),
            pltpu.VMEM((per_tile // nstage, cb), jnp.int32),
            pltpu.VMEM((nbuf, cb, width), jnp.float32),
            pltpu.VMEM_SHARED((N_PAD, width), jnp.float32),
            pltpu.SemaphoreType.DMA((nbuf,)),
            pltpu.SemaphoreType.DMA((nbuf,)),
        ],
        compiler_params=pltpu.CompilerParams(use_tc_tiling_on_sc=False),
    )


_prop128 = _make_prop_kernel(D_H, 4, 4, 64)
_prop64 = _make_prop_kernel(D_OUT, 4, 1, B)


def _k2_body(feat_ref, w1_ref, y_ref):
    y_ref[...] = jnp.dot(feat_ref[...], w1_ref[...],
                         preferred_element_type=jnp.float32)


def _k4_body(p_ref, deg_ref, b1_ref, w2_ref, z_ref):
    p = p_ref[0] + p_ref[1]
    rs_in = lax.rsqrt(jnp.maximum(deg_ref[1], 1.0))
    rs_out = lax.rsqrt(jnp.maximum(deg_ref[0], 1.0))
    h1 = jnp.maximum(p * rs_in + b1_ref[...], 0.0)
    z_ref[...] = jnp.dot(h1 * rs_out, w2_ref[...],
                         preferred_element_type=jnp.float32)


_GRID = N_PAD // RB

_k2 = pl.pallas_call(
    _k2_body,
    grid=(_GRID,),
    in_specs=[
        pl.BlockSpec((RB, D_IN), lambda i: (i, 0)),
        pl.BlockSpec((D_IN, D_H), lambda i: (0, 0)),
    ],
    out_specs=pl.BlockSpec((RB, D_H), lambda i: (i, 0)),
    out_shape=jax.ShapeDtypeStruct((N_PAD, D_H), jnp.float32),
)

_k4 = pl.pallas_call(
    _k4_body,
    grid=(_GRID,),
    in_specs=[
        pl.BlockSpec((NC, RB, D_H), lambda i: (0, i, 0)),
        pl.BlockSpec((NC, RB, 1), lambda i: (0, i, 0)),
        pl.BlockSpec((1, D_H), lambda i: (0, 0)),
        pl.BlockSpec((D_H, D_OUT), lambda i: (0, 0)),
    ],
    out_specs=pl.BlockSpec((RB, D_OUT), lambda i: (i, 0)),
    out_shape=jax.ShapeDtypeStruct((N_PAD, D_OUT), jnp.float32),
)


@jax.jit
def kernel(feat, edge_index, W1, b1, W2, b2):
    feat_pad = jnp.zeros((N_PAD, D_IN), jnp.float32).at[:N].set(feat)
    npad = E_PAD - E
    pad_idx = (N + (jnp.arange(npad, dtype=jnp.int32) % (N_PAD - N)))
    ei_pad = jnp.concatenate(
        [edge_index, jnp.broadcast_to(pad_idx, (2, npad))], axis=1
    ).reshape(2, NCHUNK, B)

    zcol = jnp.zeros((N_PAD,), jnp.float32)
    zeros128 = jnp.zeros((N_PAD, D_H), jnp.float32)
    zeros64 = jnp.zeros((N_PAD, D_OUT), jnp.float32)

    deg = _deg_kernel(ei_pad, zcol)
    deg3 = deg.reshape(NC, N_PAD, 1)

    ei64 = ei_pad.reshape(2, E_PAD // 64, 64)

    f = _k2(feat_pad, W1)
    rs_out = lax.rsqrt(jnp.maximum(deg[0], 1.0))[:, None]
    rs_in = lax.rsqrt(jnp.maximum(deg[1], 1.0))[:, None]
    y = f * rs_out
    p1 = _prop128(y, ei64, zeros128)
    z = _k4(p1, deg3, b1.reshape(1, D_H), W2)
    p2 = _prop64(z, ei_pad, zeros64)
    out = (p2[0] + p2[1]) * rs_in + b2
    return out[:N]


# speedup vs baseline: 16.1963x; 1.0173x over previous
import functools

import jax
import jax.numpy as jnp
from jax import lax
from jax.experimental import pallas as pl
from jax.experimental.pallas import tpu as pltpu
from jax.experimental.pallas import tpu_sc as plsc

N = 10000
E = 320000
D_IN = 128
D_H = 128
D_OUT = 64

NC = 2
NS = 16
B = 128
N_PAD = 10240
ROWS_PER_TILE = N_PAD // NS
E_PAD = 327680
NCHUNK = E_PAD // B
RB = 1024

_mesh = plsc.VectorSubcoreMesh(
    core_axis_name="c", subcore_axis_name="s", num_cores=NC, num_subcores=NS
)


def _deg_body(ei_hbm, zcol_hbm, deg_hbm, idx_all, ones_v, hist_sh, sem_s):
    c = lax.axis_index("c")
    s = lax.axis_index("s")
    pltpu.sync_copy(
        zcol_hbm.at[pl.ds(s * ROWS_PER_TILE, ROWS_PER_TILE)],
        hist_sh.at[pl.ds(s * ROWS_PER_TILE, ROWS_PER_TILE)],
    )
    for k in range(B // 16):
        ones_v[pl.ds(k * 16, 16)] = jnp.full((16,), 1.0, jnp.float32)
    per_tile = NCHUNK // NS
    pltpu.sync_copy(ei_hbm.at[c, pl.ds(s * per_tile, per_tile)], idx_all)
    plsc.subcore_barrier()

    GRP = 16

    @pl.loop(0, per_tile // GRP)
    def _grp(g):
        for j in range(GRP):
            pltpu.async_copy(ones_v, hist_sh.at[idx_all.at[g * GRP + j]],
                             sem_s, add=True)
        for j in range(GRP):
            pltpu.make_async_copy(
                ones_v, hist_sh.at[idx_all.at[g * GRP + j]], sem_s).wait()

    plsc.subcore_barrier()
    pltpu.sync_copy(
        hist_sh.at[pl.ds(s * ROWS_PER_TILE, ROWS_PER_TILE)],
        deg_hbm.at[c, pl.ds(s * ROWS_PER_TILE, ROWS_PER_TILE)],
    )


_deg_kernel = pl.kernel(
    _deg_body,
    out_type=jax.ShapeDtypeStruct((NC, N_PAD), jnp.float32),
    mesh=_mesh,
    scratch_types=[
        pltpu.VMEM((NCHUNK // NS, B), jnp.int32),
        pltpu.VMEM((B,), jnp.float32),
        pltpu.VMEM_SHARED((N_PAD,), jnp.float32),
        pltpu.SemaphoreType.DMA,
    ],
)


def _prop_body(width, nbuf, nstage, cb, tab_hbm, ei_hbm, zer_hbm, out_hbm,
               idx_src, idx_dst, rows_v, acc_sh, sem_g, sem_s):
    c = lax.axis_index("c")
    s = lax.axis_index("s")
    zdesc = pltpu.async_copy(
        zer_hbm.at[pl.ds(s * ROWS_PER_TILE, ROWS_PER_TILE), :],
        acc_sh.at[pl.ds(s * ROWS_PER_TILE, ROWS_PER_TILE), :],
        sem_g.at[0],
    )
    nchunk = E_PAD // cb
    per_tile = nchunk // (NC * NS)
    stage = per_tile // nstage
    base = c * (nchunk // NC) + s * per_tile

    def gather(t, b):
        pltpu.async_copy(tab_hbm.at[idx_src.at[t]], rows_v.at[b],
                         sem_g.at[b])

    def gather_wait(t, b):
        pltpu.make_async_copy(tab_hbm.at[idx_src.at[t]], rows_v.at[b],
                              sem_g.at[b]).wait()

    def scat(t, b):
        pltpu.async_copy(rows_v.at[b], acc_sh.at[idx_dst.at[t]],
                         sem_s.at[b], add=True)

    def scat_wait(t, b):
        pltpu.make_async_copy(rows_v.at[b], acc_sh.at[idx_dst.at[t]],
                              sem_s.at[b]).wait()

    ngrp = stage // nbuf

    @pl.loop(0, nstage)
    def _stage(h):
        pltpu.sync_copy(ei_hbm.at[0, pl.ds(base + h * stage, stage)], idx_src)
        pltpu.sync_copy(ei_hbm.at[1, pl.ds(base + h * stage, stage)], idx_dst)

        @pl.when(h == 0)
        def _():
            zdesc.wait()
            plsc.subcore_barrier()

        for j in range(nbuf):
            gather(j, j)

        @pl.loop(0, ngrp)
        def _grp(i):
            t0 = i * nbuf
            for j in range(nbuf):
                t = t0 + j
                jp = (j - 1) % nbuf
                gather_wait(t, j)

                @pl.when(t > 0)
                def _():
                    scat_wait(t - 1, jp)

                scat(t, j)

                @pl.when((t > 0) & (t - 1 + nbuf < stage))
                def _():
                    gather(t - 1 + nbuf, jp)

        scat_wait(stage - 1, (stage - 1) % nbuf)

    plsc.subcore_barrier()
    pltpu.sync_copy(
        acc_sh.at[pl.ds(s * ROWS_PER_TILE, ROWS_PER_TILE), :],
        out_hbm.at[c, pl.ds(s * ROWS_PER_TILE, ROWS_PER_TILE), :],
    )


def _make_prop_kernel(width, nbuf, nstage, cb):
    per_tile = (E_PAD // cb) // (NC * NS)
    return pl.kernel(
        functools.partial(_prop_body, width, nbuf, nstage, cb),
        out_type=jax.ShapeDtypeStruct((NC, N_PAD, width), jnp.float32),
        mesh=_mesh,
        scratch_types=[
            pltpu.VMEM((per_tile // nstage, cb), jnp.int32),
            pltpu.VMEM((per_tile // nstage, cb), jnp.int32),
            pltpu.VMEM((nbuf, cb, width), jnp.float32),
            pltpu.VMEM_SHARED((N_PAD, width), jnp.float32),
            pltpu.SemaphoreType.DMA((nbuf,)),
            pltpu.SemaphoreType.DMA((nbuf,)),
        ],
        compiler_params=pltpu.CompilerParams(use_tc_tiling_on_sc=False),
    )


_prop128 = _make_prop_kernel(D_H, 4, 4, 64)
_prop64 = _make_prop_kernel(D_OUT, 4, 1, B)


def _k2_body(feat_ref, w1_ref, y_ref):
    y_ref[...] = jnp.dot(feat_ref[...], w1_ref[...],
                         preferred_element_type=jnp.float32)


def _k4_body(p_ref, deg_ref, b1_ref, w2_ref, z_ref):
    p = p_ref[0] + p_ref[1]
    rs_in = lax.rsqrt(jnp.maximum(deg_ref[1], 1.0))
    rs_out = lax.rsqrt(jnp.maximum(deg_ref[0], 1.0))
    h1 = jnp.maximum(p * rs_in + b1_ref[...], 0.0)
    z_ref[...] = jnp.dot(h1 * rs_out, w2_ref[...],
                         preferred_element_type=jnp.float32)


_GRID = N_PAD // RB

_k2 = pl.pallas_call(
    _k2_body,
    grid=(_GRID,),
    in_specs=[
        pl.BlockSpec((RB, D_IN), lambda i: (i, 0)),
        pl.BlockSpec((D_IN, D_H), lambda i: (0, 0)),
    ],
    out_specs=pl.BlockSpec((RB, D_H), lambda i: (i, 0)),
    out_shape=jax.ShapeDtypeStruct((N_PAD, D_H), jnp.float32),
)

_k4 = pl.pallas_call(
    _k4_body,
    grid=(_GRID,),
    in_specs=[
        pl.BlockSpec((NC, RB, D_H), lambda i: (0, i, 0)),
        pl.BlockSpec((NC, RB, 1), lambda i: (0, i, 0)),
        pl.BlockSpec((1, D_H), lambda i: (0, 0)),
        pl.BlockSpec((D_H, D_OUT), lambda i: (0, 0)),
    ],
    out_specs=pl.BlockSpec((RB, D_OUT), lambda i: (i, 0)),
    out_shape=jax.ShapeDtypeStruct((N_PAD, D_OUT), jnp.float32),
)


@jax.jit
def kernel(feat, edge_index, W1, b1, W2, b2):
    feat_pad = jnp.zeros((N_PAD, D_IN), jnp.float32).at[:N].set(feat)
    npad = E_PAD - E
    pad_idx = (N + (jnp.arange(npad, dtype=jnp.int32) % (N_PAD - N)))
    ei_pad = jnp.concatenate(
        [edge_index, jnp.broadcast_to(pad_idx, (2, npad))], axis=1
    ).reshape(2, NCHUNK, B)

    zcol = jnp.zeros((N_PAD,), jnp.float32)
    zeros128 = jnp.zeros((N_PAD, D_H), jnp.float32)
    zeros64 = jnp.zeros((N_PAD, D_OUT), jnp.float32)

    deg = _deg_kernel(ei_pad, zcol)
    deg3 = deg.reshape(NC, N_PAD, 1)

    ei64 = ei_pad.reshape(2, E_PAD // 64, 64)

    f = _k2(feat_pad, W1)
    rs_out = lax.rsqrt(jnp.maximum(deg[0], 1.0))[:, None]
    rs_in = lax.rsqrt(jnp.maximum(deg[1], 1.0))[:, None]
    y = f * rs_out
    p1 = _prop128(y, ei64, zeros128)
    z = _k4(p1, deg3, b1.reshape(1, D_H), W2)
    p2 = _prop64(z, ei_pad, zeros64)
    out = (p2[0] + p2[1]) * rs_in + b2
    return out[:N]
